# baseline (speedup 1.0000x reference)
import functools

import jax
import jax.numpy as jnp
from jax import lax
from jax.experimental import pallas as pl
from jax.experimental.pallas import tpu as pltpu
from jax.experimental.pallas import tpu_sc as plsc

N_NODES_ = 10000
N_EDGES_ = 320000
NP = 10240
NCORES = 2
NSUB = 16
NTILES = NCORES * NSUB
EDGES_PER_TILE = 10240
EPAD = EDGES_PER_TILE * NTILES
BATCH = 128
NBATCH = EDGES_PER_TILE // BATCH
ROWS_PER_TILE = NP // NSUB

_MESH = plsc.VectorSubcoreMesh(core_axis_name="c", subcore_axis_name="s")


@functools.partial(
    pl.kernel,
    out_type=[jax.ShapeDtypeStruct((NP,), jnp.float32),
              jax.ShapeDtypeStruct((NP,), jnp.float32)],
    mesh=_MESH,
    scratch_types=[
        pltpu.VMEM_SHARED((NSUB, NP), jnp.float32),
        pltpu.VMEM((NBATCH, BATCH), jnp.int32),
        pltpu.VMEM((NP,), jnp.float32),
        pltpu.VMEM((ROWS_PER_TILE,), jnp.float32),
        pltpu.VMEM((ROWS_PER_TILE,), jnp.float32),
    ],
    compiler_params=pltpu.CompilerParams(needs_layout_passes=False),
)
def _deg_kernel(dst_hbm, out0, out1, sh, dst_v, hist, red_a, red_b):
    c = lax.axis_index("c")
    s = lax.axis_index("s")
    wid = c * NSUB + s
    pltpu.sync_copy(dst_hbm.at[pl.ds(wid * NBATCH, NBATCH)], dst_v)

    zero16 = jnp.zeros((16,), jnp.float32)
    one16 = jnp.ones((16,), jnp.float32)

    def zstep(i, carry):
        hist[pl.ds(i * 16, 16)] = zero16
        return carry

    lax.fori_loop(0, NP // 16, zstep, 0)

    def hstep(j, carry):
        for k in range(BATCH // 16):
            idx = dst_v[j, pl.ds(k * 16, 16)]
            plsc.addupdate_scatter(hist, [idx], one16)
        return carry

    lax.fori_loop(0, NBATCH, hstep, 0)

    pltpu.sync_copy(hist, sh.at[s])
    plsc.subcore_barrier()

    sl = pl.ds(ROWS_PER_TILE * s, ROWS_PER_TILE)
    pltpu.sync_copy(sh.at[0].at[sl], red_a)
    for k in range(1, NSUB):
        pltpu.sync_copy(sh.at[k].at[sl], red_b)

        def astep(m, carry):
            red_a[pl.ds(m * 16, 16)] = (red_a[pl.ds(m * 16, 16)]
                                        + red_b[pl.ds(m * 16, 16)])
            return carry

        lax.fori_loop(0, ROWS_PER_TILE // 16, astep, 0)

    @pl.when(c == 0)
    def _():
        pltpu.sync_copy(red_a, out0.at[sl])

    @pl.when(c == 1)
    def _():
        pltpu.sync_copy(red_a, out1.at[sl])


SBATCH = 128
NBTOT = EPAD // SBATCH
NST0, NST1 = 5, 5
ST = 16
NBUF = 2
assert (NST0 + NST1) * ST * NSUB == NBTOT


def _make_agg(feat):
    @functools.partial(
        pl.kernel,
        out_type=jax.ShapeDtypeStruct((NCORES, NP, feat), jnp.float32),
        mesh=_MESH,
        scratch_types=[
            pltpu.VMEM_SHARED((NP, feat), jnp.float32),
            pltpu.VMEM((ST, SBATCH), jnp.int32),
            pltpu.VMEM((ST, SBATCH), jnp.int32),
            [pltpu.VMEM((SBATCH, feat), jnp.float32) for _ in range(NBUF)],
            [pltpu.SemaphoreType.DMA for _ in range(NBUF)],
        ],
    )
    def agg(x_hbm, src_hbm, dst_hbm, out_hbm,
            acc, src_v, dst_v, bufs, sems):
        c = lax.axis_index("c")
        s = lax.axis_index("s")

        zero16 = jnp.zeros((16,), jnp.float32)

        def zfill(i, carry):
            r = i // (feat // 16)
            k = i % (feat // 16)
            bufs[0][r, pl.ds(k * 16, 16)] = zero16
            return carry

        lax.fori_loop(0, SBATCH * feat // 16, zfill, 0)
        for r in range(ROWS_PER_TILE // SBATCH):
            pltpu.sync_copy(
                bufs[0],
                acc.at[pl.ds(ROWS_PER_TILE * s + r * SBATCH, SBATCH)])
        plsc.subcore_barrier()

        nst = jnp.where(c == 0, NST0, NST1)
        row0 = jnp.where(c == 0, ST * NST0 * s,
                         ST * NST0 * NSUB + ST * NST1 * s)

        def stage(h, carry):
            base = row0 + h * ST
            pltpu.sync_copy(src_hbm.at[pl.ds(base, ST)], src_v)
            pltpu.sync_copy(dst_hbm.at[pl.ds(base, ST)], dst_v)
            for q in range(NBUF - 1):
                pltpu.async_copy(x_hbm.at[src_v.at[q]], bufs[q], sems[q])

            def step(i, carry2):
                for q in range(NBUF):
                    j = NBUF * i + q
                    pltpu.make_async_copy(
                        x_hbm.at[src_v.at[j]], bufs[q], sems[q]).wait()
                    pltpu.sync_copy(bufs[q], acc.at[dst_v.at[j]], add=True)
                    qn = (q + NBUF - 1) % NBUF

                    @pl.when(j + NBUF - 1 < ST)
                    def _():
                        pltpu.async_copy(
                            x_hbm.at[src_v.at[j + NBUF - 1]],
                            bufs[qn], sems[qn])
                return carry2

            lax.fori_loop(0, ST // NBUF, step, 0)
            return carry

        lax.fori_loop(0, nst, stage, 0)
        plsc.subcore_barrier()
        sl = pl.ds(ROWS_PER_TILE * s, ROWS_PER_TILE)
        pltpu.sync_copy(acc.at[sl], out_hbm.at[c].at[sl])

    return agg


_agg128 = _make_agg(128)


BLK = 1024
GRID = NP // BLK


def _scale_body(deg0_ref, deg1_ref, x_ref, dinv_ref, xt_ref):
    d = deg0_ref[...] + deg1_ref[...] + 1.0
    di = lax.rsqrt(d)
    dinv_ref[...] = di
    xt_ref[...] = x_ref[...] * di


def _scale_call(deg0, deg1, x_pad):
    return pl.pallas_call(
        _scale_body,
        grid=(GRID,),
        in_specs=[
            pl.BlockSpec((BLK, 1), lambda i: (i, 0)),
            pl.BlockSpec((BLK, 1), lambda i: (i, 0)),
            pl.BlockSpec((BLK, 128), lambda i: (i, 0)),
        ],
        out_specs=[
            pl.BlockSpec((BLK, 1), lambda i: (i, 0)),
            pl.BlockSpec((BLK, 128), lambda i: (i, 0)),
        ],
        out_shape=[
            jax.ShapeDtypeStruct((NP, 1), jnp.float32),
            jax.ShapeDtypeStruct((NP, 128), jnp.float32),
        ],
    )(deg0, deg1, x_pad)


def _fused_body(acc_ref, xt_ref, dinv_ref, w1_ref, b1_ref, w2_ref, out_ref):
    di = dinv_ref[...]
    z = (acc_ref[0] + acc_ref[1] + xt_ref[...]) * di
    h = jnp.dot(z, w1_ref[...], preferred_element_type=jnp.float32)
    h = jnp.maximum(h + b1_ref[...], 0.0)
    t = jnp.dot(h, w2_ref[...], preferred_element_type=jnp.float32)
    out_ref[...] = jnp.concatenate(
        [t * di, jnp.zeros((t.shape[0], 64), jnp.float32)], axis=1)


def _fused_call(acc1, xt, dinv, W1, b1, W2):
    return pl.pallas_call(
        _fused_body,
        grid=(GRID,),
        in_specs=[
            pl.BlockSpec((NCORES, BLK, 128), lambda i: (0, i, 0)),
            pl.BlockSpec((BLK, 128), lambda i: (i, 0)),
            pl.BlockSpec((BLK, 1), lambda i: (i, 0)),
            pl.BlockSpec((128, 1024), lambda i: (0, 0)),
            pl.BlockSpec((1, 1024), lambda i: (0, 0)),
            pl.BlockSpec((1024, 64), lambda i: (0, 0)),
        ],
        out_specs=pl.BlockSpec((BLK, 128), lambda i: (i, 0)),
        out_shape=jax.ShapeDtypeStruct((NP, 128), jnp.float32),
    )(acc1, xt, dinv, W1, b1.reshape(1, 1024), W2)


def _softmax_body(acc_ref, tt_ref, dinv_ref, b2_ref, out_ref):
    z128 = (acc_ref[0] + acc_ref[1] + tt_ref[...]) * dinv_ref[...]
    z = z128[:, :64] + b2_ref[...]
    m = jnp.max(z, axis=1, keepdims=True)
    e = jnp.exp(z - m)
    out_ref[...] = e / jnp.sum(e, axis=1, keepdims=True)


def _softmax_call(acc2, tt, dinv, b2):
    return pl.pallas_call(
        _softmax_body,
        grid=(GRID,),
        in_specs=[
            pl.BlockSpec((NCORES, BLK, 128), lambda i: (0, i, 0)),
            pl.BlockSpec((BLK, 128), lambda i: (i, 0)),
            pl.BlockSpec((BLK, 1), lambda i: (i, 0)),
            pl.BlockSpec((1, 64), lambda i: (0, 0)),
        ],
        out_specs=pl.BlockSpec((BLK, 64), lambda i: (i, 0)),
        out_shape=jax.ShapeDtypeStruct((NP, 64), jnp.float32),
    )(acc2, tt, dinv, b2.reshape(1, 64))


def kernel(x, edge_index, W1, b1, W2, b2):
    n = x.shape[0]
    e = edge_index.shape[1]
    src = edge_index[0].astype(jnp.int32)
    dst = edge_index[1].astype(jnp.int32)
    pad = EPAD - e
    pad_idx = n + (jnp.arange(pad, dtype=jnp.int32) % (NP - n))
    src_p = jnp.concatenate([src, pad_idx])
    dst_p = jnp.concatenate([dst, pad_idx])
    src2d = src_p.reshape(EPAD // BATCH, BATCH)
    dst2d = dst_p.reshape(EPAD // BATCH, BATCH)
    src64 = src_p.reshape(EPAD // SBATCH, SBATCH)
    dst64 = dst_p.reshape(EPAD // SBATCH, SBATCH)
    x_pad = jnp.zeros((NP, 128), jnp.float32).at[:n].set(x)

    deg0, deg1 = _deg_kernel(dst2d)
    dinv, xt = _scale_call(deg0.reshape(NP, 1), deg1.reshape(NP, 1), x_pad)
    acc1 = _agg128(xt, src64, dst64)
    tt = _fused_call(acc1, xt, dinv, W1, b1, W2)
    acc2 = _agg128(tt, src64, dst64)
    out = _softmax_call(acc2, tt, dinv, b2)
    return out[:n]


# speedup vs baseline: 1.4126x; 1.4126x over previous
import functools

import jax
import jax.numpy as jnp
from jax import lax
from jax.experimental import pallas as pl
from jax.experimental.pallas import tpu as pltpu
from jax.experimental.pallas import tpu_sc as plsc

N_NODES_ = 10000
N_EDGES_ = 320000
NP = 10240
NCORES = 2
NSUB = 16
NTILES = NCORES * NSUB
EDGES_PER_TILE = 10240
EPAD = EDGES_PER_TILE * NTILES
BATCH = 128
NBATCH = EDGES_PER_TILE // BATCH
ROWS_PER_TILE = NP // NSUB

_MESH = plsc.VectorSubcoreMesh(core_axis_name="c", subcore_axis_name="s")


@functools.partial(
    pl.kernel,
    out_type=[jax.ShapeDtypeStruct((NP,), jnp.float32),
              jax.ShapeDtypeStruct((NP,), jnp.float32)],
    mesh=_MESH,
    scratch_types=[
        pltpu.VMEM_SHARED((NSUB, NP), jnp.float32),
        pltpu.VMEM((NBATCH, BATCH), jnp.int32),
        pltpu.VMEM((NP,), jnp.float32),
        pltpu.VMEM((ROWS_PER_TILE,), jnp.float32),
        pltpu.VMEM((ROWS_PER_TILE,), jnp.float32),
    ],
    compiler_params=pltpu.CompilerParams(needs_layout_passes=False),
)
def _deg_kernel(dst_hbm, out0, out1, sh, dst_v, hist, red_a, red_b):
    c = lax.axis_index("c")
    s = lax.axis_index("s")
    wid = c * NSUB + s
    pltpu.sync_copy(dst_hbm.at[pl.ds(wid * NBATCH, NBATCH)], dst_v)

    zero16 = jnp.zeros((16,), jnp.float32)
    one16 = jnp.ones((16,), jnp.float32)

    def zstep(i, carry):
        hist[pl.ds(i * 16, 16)] = zero16
        return carry

    lax.fori_loop(0, NP // 16, zstep, 0)

    def hstep(j, carry):
        for k in range(BATCH // 16):
            idx = dst_v[j, pl.ds(k * 16, 16)]
            plsc.addupdate_scatter(hist, [idx], one16)
        return carry

    lax.fori_loop(0, NBATCH, hstep, 0)

    pltpu.sync_copy(hist, sh.at[s])
    plsc.subcore_barrier()

    sl = pl.ds(ROWS_PER_TILE * s, ROWS_PER_TILE)
    pltpu.sync_copy(sh.at[0].at[sl], red_a)
    for k in range(1, NSUB):
        pltpu.sync_copy(sh.at[k].at[sl], red_b)

        def astep(m, carry):
            red_a[pl.ds(m * 16, 16)] = (red_a[pl.ds(m * 16, 16)]
                                        + red_b[pl.ds(m * 16, 16)])
            return carry

        lax.fori_loop(0, ROWS_PER_TILE // 16, astep, 0)

    @pl.when(c == 0)
    def _():
        pltpu.sync_copy(red_a, out0.at[sl])

    @pl.when(c == 1)
    def _():
        pltpu.sync_copy(red_a, out1.at[sl])


SBATCH = 64
NBTOT = EPAD // SBATCH
NST0, NST1 = 5, 5
ST = 32
NBUF = 4
assert (NST0 + NST1) * ST * NSUB == NBTOT


def _make_agg(feat):
    @functools.partial(
        pl.kernel,
        out_type=jax.ShapeDtypeStruct((NCORES, NP, feat), jnp.float32),
        mesh=_MESH,
        scratch_types=[
            pltpu.VMEM_SHARED((NP, feat), jnp.float32),
            pltpu.VMEM((ST, SBATCH), jnp.int32),
            pltpu.VMEM((ST, SBATCH), jnp.int32),
            [pltpu.VMEM((SBATCH, feat), jnp.float32) for _ in range(NBUF)],
            [pltpu.SemaphoreType.DMA for _ in range(NBUF)],
            [pltpu.SemaphoreType.DMA for _ in range(NBUF)],
        ],
    )
    def agg(x_hbm, src_hbm, dst_hbm, out_hbm,
            acc, src_v, dst_v, bufs, sems, ssems):
        c = lax.axis_index("c")
        s = lax.axis_index("s")

        zero16 = jnp.zeros((16,), jnp.float32)

        def zfill(i, carry):
            r = i // (feat // 16)
            k = i % (feat // 16)
            bufs[0][r, pl.ds(k * 16, 16)] = zero16
            return carry

        lax.fori_loop(0, SBATCH * feat // 16, zfill, 0)
        for r in range(ROWS_PER_TILE // SBATCH):
            pltpu.sync_copy(
                bufs[0],
                acc.at[pl.ds(ROWS_PER_TILE * s + r * SBATCH, SBATCH)])
        plsc.subcore_barrier()

        nst = jnp.where(c == 0, NST0, NST1)
        row0 = jnp.where(c == 0, ST * NST0 * s,
                         ST * NST0 * NSUB + ST * NST1 * s)

        def stage(h, carry):
            base = row0 + h * ST
            pltpu.sync_copy(src_hbm.at[pl.ds(base, ST)], src_v)
            pltpu.sync_copy(dst_hbm.at[pl.ds(base, ST)], dst_v)
            for q in range(NBUF - 1):
                pltpu.async_copy(x_hbm.at[src_v.at[q]], bufs[q], sems[q])

            def step(i, carry2):
                for q in range(NBUF):
                    j = NBUF * i + q
                    pltpu.make_async_copy(
                        x_hbm.at[src_v.at[j]], bufs[q], sems[q]).wait()
                    pltpu.async_copy(
                        bufs[q], acc.at[dst_v.at[j]], ssems[q], add=True)
                    qn = (q + NBUF - 1) % NBUF

                    @pl.when(j + NBUF - 1 < ST)
                    def _():
                        @pl.when(j > 0)
                        def _():
                            pltpu.make_async_copy(
                                bufs[qn], acc.at[dst_v.at[j - 1]],
                                ssems[qn]).wait()

                        pltpu.async_copy(
                            x_hbm.at[src_v.at[j + NBUF - 1]],
                            bufs[qn], sems[qn])
                return carry2

            lax.fori_loop(0, ST // NBUF, step, 0)
            for jj in range(ST - NBUF, ST):
                q = jj % NBUF
                pltpu.make_async_copy(
                    bufs[q], acc.at[dst_v.at[jj]], ssems[q]).wait()
            return carry

        lax.fori_loop(0, nst, stage, 0)
        plsc.subcore_barrier()
        sl = pl.ds(ROWS_PER_TILE * s, ROWS_PER_TILE)
        pltpu.sync_copy(acc.at[sl], out_hbm.at[c].at[sl])

    return agg


_agg128 = _make_agg(128)


BLK = 1024
GRID = NP // BLK


def _scale_body(deg0_ref, deg1_ref, x_ref, dinv_ref, xt_ref):
    d = deg0_ref[...] + deg1_ref[...] + 1.0
    di = lax.rsqrt(d)
    dinv_ref[...] = di
    xt_ref[...] = x_ref[...] * di


def _scale_call(deg0, deg1, x_pad):
    return pl.pallas_call(
        _scale_body,
        grid=(GRID,),
        in_specs=[
            pl.BlockSpec((BLK, 1), lambda i: (i, 0)),
            pl.BlockSpec((BLK, 1), lambda i: (i, 0)),
            pl.BlockSpec((BLK, 128), lambda i: (i, 0)),
        ],
        out_specs=[
            pl.BlockSpec((BLK, 1), lambda i: (i, 0)),
            pl.BlockSpec((BLK, 128), lambda i: (i, 0)),
        ],
        out_shape=[
            jax.ShapeDtypeStruct((NP, 1), jnp.float32),
            jax.ShapeDtypeStruct((NP, 128), jnp.float32),
        ],
    )(deg0, deg1, x_pad)


def _fused_body(acc_ref, xt_ref, dinv_ref, w1_ref, b1_ref, w2_ref, out_ref):
    di = dinv_ref[...]
    z = (acc_ref[0] + acc_ref[1] + xt_ref[...]) * di
    h = jnp.dot(z, w1_ref[...], preferred_element_type=jnp.float32)
    h = jnp.maximum(h + b1_ref[...], 0.0)
    t = jnp.dot(h, w2_ref[...], preferred_element_type=jnp.float32)
    out_ref[...] = jnp.concatenate(
        [t * di, jnp.zeros((t.shape[0], 64), jnp.float32)], axis=1)


def _fused_call(acc1, xt, dinv, W1, b1, W2):
    return pl.pallas_call(
        _fused_body,
        grid=(GRID,),
        in_specs=[
            pl.BlockSpec((NCORES, BLK, 128), lambda i: (0, i, 0)),
            pl.BlockSpec((BLK, 128), lambda i: (i, 0)),
            pl.BlockSpec((BLK, 1), lambda i: (i, 0)),
            pl.BlockSpec((128, 1024), lambda i: (0, 0)),
            pl.BlockSpec((1, 1024), lambda i: (0, 0)),
            pl.BlockSpec((1024, 64), lambda i: (0, 0)),
        ],
        out_specs=pl.BlockSpec((BLK, 128), lambda i: (i, 0)),
        out_shape=jax.ShapeDtypeStruct((NP, 128), jnp.float32),
    )(acc1, xt, dinv, W1, b1.reshape(1, 1024), W2)


def _softmax_body(acc_ref, tt_ref, dinv_ref, b2_ref, out_ref):
    z128 = (acc_ref[0] + acc_ref[1] + tt_ref[...]) * dinv_ref[...]
    z = z128[:, :64] + b2_ref[...]
    m = jnp.max(z, axis=1, keepdims=True)
    e = jnp.exp(z - m)
    out_ref[...] = e / jnp.sum(e, axis=1, keepdims=True)


def _softmax_call(acc2, tt, dinv, b2):
    return pl.pallas_call(
        _softmax_body,
        grid=(GRID,),
        in_specs=[
            pl.BlockSpec((NCORES, BLK, 128), lambda i: (0, i, 0)),
            pl.BlockSpec((BLK, 128), lambda i: (i, 0)),
            pl.BlockSpec((BLK, 1), lambda i: (i, 0)),
            pl.BlockSpec((1, 64), lambda i: (0, 0)),
        ],
        out_specs=pl.BlockSpec((BLK, 64), lambda i: (i, 0)),
        out_shape=jax.ShapeDtypeStruct((NP, 64), jnp.float32),
    )(acc2, tt, dinv, b2.reshape(1, 64))


def kernel(x, edge_index, W1, b1, W2, b2):
    n = x.shape[0]
    e = edge_index.shape[1]
    src = edge_index[0].astype(jnp.int32)
    dst = edge_index[1].astype(jnp.int32)
    pad = EPAD - e
    pad_idx = n + (jnp.arange(pad, dtype=jnp.int32) % (NP - n))
    src_p = jnp.concatenate([src, pad_idx])
    dst_p = jnp.concatenate([dst, pad_idx])
    src2d = src_p.reshape(EPAD // BATCH, BATCH)
    dst2d = dst_p.reshape(EPAD // BATCH, BATCH)
    src64 = src_p.reshape(EPAD // SBATCH, SBATCH)
    dst64 = dst_p.reshape(EPAD // SBATCH, SBATCH)
    x_pad = jnp.zeros((NP, 128), jnp.float32).at[:n].set(x)

    deg0, deg1 = _deg_kernel(dst2d)
    dinv, xt = _scale_call(deg0.reshape(NP, 1), deg1.reshape(NP, 1), x_pad)
    acc1 = _agg128(xt, src64, dst64)
    tt = _fused_call(acc1, xt, dinv, W1, b1, W2)
    acc2 = _agg128(tt, src64, dst64)
    out = _softmax_call(acc2, tt, dinv, b2)
    return out[:n]


# speedup vs baseline: 1.4310x; 1.0131x over previous
import functools

import jax
import jax.numpy as jnp
from jax import lax
from jax.experimental import pallas as pl
from jax.experimental.pallas import tpu as pltpu
from jax.experimental.pallas import tpu_sc as plsc

N_NODES_ = 10000
N_EDGES_ = 320000
NP = 10240
NCORES = 2
NSUB = 16
NTILES = NCORES * NSUB
EDGES_PER_TILE = 10240
EPAD = EDGES_PER_TILE * NTILES
BATCH = 128
NBATCH = EDGES_PER_TILE // BATCH
ROWS_PER_TILE = NP // NSUB

_MESH = plsc.VectorSubcoreMesh(core_axis_name="c", subcore_axis_name="s")


@functools.partial(
    pl.kernel,
    out_type=[jax.ShapeDtypeStruct((NP,), jnp.float32),
              jax.ShapeDtypeStruct((NP,), jnp.float32)],
    mesh=_MESH,
    scratch_types=[
        pltpu.VMEM_SHARED((NSUB, NP), jnp.float32),
        pltpu.VMEM((160, 64), jnp.int32),
        pltpu.VMEM((NP,), jnp.float32),
        pltpu.VMEM((ROWS_PER_TILE,), jnp.float32),
        pltpu.VMEM((ROWS_PER_TILE,), jnp.float32),
    ],
    compiler_params=pltpu.CompilerParams(needs_layout_passes=False),
)
def _deg_kernel(dst_hbm, out0, out1, sh, dst_v, hist, red_a, red_b):
    c = lax.axis_index("c")
    s = lax.axis_index("s")
    wid = c * NSUB + s
    pltpu.sync_copy(dst_hbm.at[pl.ds(wid * 160, 160)], dst_v)

    zero16 = jnp.zeros((16,), jnp.float32)
    one16 = jnp.ones((16,), jnp.float32)

    def zstep(i, carry):
        hist[pl.ds(i * 16, 16)] = zero16
        return carry

    lax.fori_loop(0, NP // 16, zstep, 0)

    def hstep(j, carry):
        for k in range(64 // 16):
            idx = dst_v[j, pl.ds(k * 16, 16)]
            plsc.addupdate_scatter(hist, [idx], one16)
        return carry

    lax.fori_loop(0, 160, hstep, 0)

    pltpu.sync_copy(hist, sh.at[s])
    plsc.subcore_barrier()

    sl = pl.ds(ROWS_PER_TILE * s, ROWS_PER_TILE)
    pltpu.sync_copy(sh.at[0].at[sl], red_a)
    for k in range(1, NSUB):
        pltpu.sync_copy(sh.at[k].at[sl], red_b)

        def astep(m, carry):
            red_a[pl.ds(m * 16, 16)] = (red_a[pl.ds(m * 16, 16)]
                                        + red_b[pl.ds(m * 16, 16)])
            return carry

        lax.fori_loop(0, ROWS_PER_TILE // 16, astep, 0)

    @pl.when(c == 0)
    def _():
        pltpu.sync_copy(red_a, out0.at[sl])

    @pl.when(c == 1)
    def _():
        pltpu.sync_copy(red_a, out1.at[sl])


SBATCH = 64
NBTOT = EPAD // SBATCH
NST0, NST1 = 5, 5
ST = 32
NBUF = 4
assert (NST0 + NST1) * ST * NSUB == NBTOT


def _make_agg(feat):
    @functools.partial(
        pl.kernel,
        out_type=jax.ShapeDtypeStruct((NCORES, NP, feat), jnp.float32),
        mesh=_MESH,
        scratch_types=[
            pltpu.VMEM_SHARED((NP, feat), jnp.float32),
            pltpu.VMEM((ST, SBATCH), jnp.int32),
            pltpu.VMEM((ST, SBATCH), jnp.int32),
            [pltpu.VMEM((SBATCH, feat), jnp.float32) for _ in range(NBUF)],
            [pltpu.SemaphoreType.DMA for _ in range(NBUF)],
        ],
    )
    def agg(x_hbm, src_hbm, dst_hbm, out_hbm,
            acc, src_v, dst_v, bufs, sems):
        c = lax.axis_index("c")
        s = lax.axis_index("s")

        zero16 = jnp.zeros((16,), jnp.float32)

        def zfill(i, carry):
            r = i // (feat // 16)
            k = i % (feat // 16)
            bufs[0][r, pl.ds(k * 16, 16)] = zero16
            return carry

        lax.fori_loop(0, SBATCH * feat // 16, zfill, 0)
        for r in range(ROWS_PER_TILE // SBATCH):
            pltpu.sync_copy(
                bufs[0],
                acc.at[pl.ds(ROWS_PER_TILE * s + r * SBATCH, SBATCH)])
        plsc.subcore_barrier()

        nst = jnp.where(c == 0, NST0, NST1)
        row0 = jnp.where(c == 0, ST * NST0 * s,
                         ST * NST0 * NSUB + ST * NST1 * s)

        def stage(h, carry):
            base = row0 + h * ST
            pltpu.sync_copy(src_hbm.at[pl.ds(base, ST)], src_v)
            pltpu.sync_copy(dst_hbm.at[pl.ds(base, ST)], dst_v)
            for q in range(NBUF - 1):
                pltpu.async_copy(x_hbm.at[src_v.at[q]], bufs[q], sems[q])

            def step(i, carry2):
                for q in range(NBUF):
                    j = NBUF * i + q
                    pltpu.make_async_copy(
                        x_hbm.at[src_v.at[j]], bufs[q], sems[q]).wait()
                    pltpu.sync_copy(bufs[q], acc.at[dst_v.at[j]], add=True)
                    qn = (q + NBUF - 1) % NBUF

                    @pl.when(j + NBUF - 1 < ST)
                    def _():
                        pltpu.async_copy(
                            x_hbm.at[src_v.at[j + NBUF - 1]],
                            bufs[qn], sems[qn])
                return carry2

            lax.fori_loop(0, ST // NBUF, step, 0)
            return carry

        lax.fori_loop(0, nst, stage, 0)
        plsc.subcore_barrier()
        sl = pl.ds(ROWS_PER_TILE * s, ROWS_PER_TILE)
        pltpu.sync_copy(acc.at[sl], out_hbm.at[c].at[sl])

    return agg


_agg128 = _make_agg(128)


BLK = 1024
GRID = NP // BLK


def _scale_body(deg0_ref, deg1_ref, x_ref, dinv_ref, xt_ref):
    d = deg0_ref[...] + deg1_ref[...] + 1.0
    di = lax.rsqrt(d)
    dinv_ref[...] = di
    xt_ref[...] = x_ref[...] * di


def _scale_call(deg0, deg1, x_pad):
    return pl.pallas_call(
        _scale_body,
        grid=(GRID,),
        in_specs=[
            pl.BlockSpec((BLK, 1), lambda i: (i, 0)),
            pl.BlockSpec((BLK, 1), lambda i: (i, 0)),
            pl.BlockSpec((BLK, 128), lambda i: (i, 0)),
        ],
        out_specs=[
            pl.BlockSpec((BLK, 1), lambda i: (i, 0)),
            pl.BlockSpec((BLK, 128), lambda i: (i, 0)),
        ],
        out_shape=[
            jax.ShapeDtypeStruct((NP, 1), jnp.float32),
            jax.ShapeDtypeStruct((NP, 128), jnp.float32),
        ],
    )(deg0, deg1, x_pad)


def _fused_body(acc_ref, xt_ref, dinv_ref, w1_ref, b1_ref, w2_ref, out_ref):
    di = dinv_ref[...]
    z = (acc_ref[0] + acc_ref[1] + xt_ref[...]) * di
    h = jnp.dot(z, w1_ref[...], preferred_element_type=jnp.float32)
    h = jnp.maximum(h + b1_ref[...], 0.0)
    t = jnp.dot(h, w2_ref[...], preferred_element_type=jnp.float32)
    out_ref[...] = jnp.concatenate(
        [t * di, jnp.zeros((t.shape[0], 64), jnp.float32)], axis=1)


def _fused_call(acc1, xt, dinv, W1, b1, W2):
    return pl.pallas_call(
        _fused_body,
        grid=(GRID,),
        in_specs=[
            pl.BlockSpec((NCORES, BLK, 128), lambda i: (0, i, 0)),
            pl.BlockSpec((BLK, 128), lambda i: (i, 0)),
            pl.BlockSpec((BLK, 1), lambda i: (i, 0)),
            pl.BlockSpec((128, 1024), lambda i: (0, 0)),
            pl.BlockSpec((1, 1024), lambda i: (0, 0)),
            pl.BlockSpec((1024, 64), lambda i: (0, 0)),
        ],
        out_specs=pl.BlockSpec((BLK, 128), lambda i: (i, 0)),
        out_shape=jax.ShapeDtypeStruct((NP, 128), jnp.float32),
    )(acc1, xt, dinv, W1, b1.reshape(1, 1024), W2)


def _softmax_body(acc_ref, tt_ref, dinv_ref, b2_ref, out_ref):
    z128 = (acc_ref[0] + acc_ref[1] + tt_ref[...]) * dinv_ref[...]
    z = z128[:, :64] + b2_ref[...]
    m = jnp.max(z, axis=1, keepdims=True)
    e = jnp.exp(z - m)
    out_ref[...] = e / jnp.sum(e, axis=1, keepdims=True)


def _softmax_call(acc2, tt, dinv, b2):
    return pl.pallas_call(
        _softmax_body,
        grid=(GRID,),
        in_specs=[
            pl.BlockSpec((NCORES, BLK, 128), lambda i: (0, i, 0)),
            pl.BlockSpec((BLK, 128), lambda i: (i, 0)),
            pl.BlockSpec((BLK, 1), lambda i: (i, 0)),
            pl.BlockSpec((1, 64), lambda i: (0, 0)),
        ],
        out_specs=pl.BlockSpec((BLK, 64), lambda i: (i, 0)),
        out_shape=jax.ShapeDtypeStruct((NP, 64), jnp.float32),
    )(acc2, tt, dinv, b2.reshape(1, 64))


def kernel(x, edge_index, W1, b1, W2, b2):
    n = x.shape[0]
    e = edge_index.shape[1]
    src = edge_index[0].astype(jnp.int32)
    dst = edge_index[1].astype(jnp.int32)
    pad = EPAD - e
    pad_idx = n + (jnp.arange(pad, dtype=jnp.int32) % (NP - n))
    src_p = jnp.concatenate([src, pad_idx])
    dst_p = jnp.concatenate([dst, pad_idx])
    src64 = src_p.reshape(EPAD // SBATCH, SBATCH)
    dst64 = dst_p.reshape(EPAD // SBATCH, SBATCH)
    x_pad = jnp.zeros((NP, 128), jnp.float32).at[:n].set(x)

    deg0, deg1 = _deg_kernel(dst64)
    dinv, xt = _scale_call(deg0.reshape(NP, 1), deg1.reshape(NP, 1), x_pad)
    acc1 = _agg128(xt, src64, dst64)
    tt = _fused_call(acc1, xt, dinv, W1, b1, W2)
    acc2 = _agg128(tt, src64, dst64)
    out = _softmax_call(acc2, tt, dinv, b2)
    return out[:n]


# speedup vs baseline: 1.4551x; 1.0168x over previous
import functools

import jax
import jax.numpy as jnp
from jax import lax
from jax.experimental import pallas as pl
from jax.experimental.pallas import tpu as pltpu
from jax.experimental.pallas import tpu_sc as plsc

N_NODES_ = 10000
N_EDGES_ = 320000
NP = 10240
NCORES = 2
NSUB = 16
NTILES = NCORES * NSUB
EDGES_PER_TILE = 10240
EPAD = EDGES_PER_TILE * NTILES
BATCH = 128
NBATCH = EDGES_PER_TILE // BATCH
ROWS_PER_TILE = NP // NSUB

_MESH = plsc.VectorSubcoreMesh(core_axis_name="c", subcore_axis_name="s")

SBATCH = 32
NBTOT = EPAD // SBATCH
NBT = EDGES_PER_TILE // SBATCH
NST0, NST1 = 5, 5
ST = NBT // (NST0 + NST1) * 2
NBUF = 8
assert (NST0 + NST1) * ST * NSUB == NBTOT and ST % NBUF == 0


@functools.partial(
    pl.kernel,
    out_type=[jax.ShapeDtypeStruct((NP,), jnp.float32),
              jax.ShapeDtypeStruct((NP,), jnp.float32)],
    mesh=_MESH,
    scratch_types=[
        pltpu.VMEM_SHARED((NSUB, NP), jnp.float32),
        pltpu.VMEM((NBT, SBATCH), jnp.int32),
        pltpu.VMEM((NP,), jnp.float32),
        pltpu.VMEM((ROWS_PER_TILE,), jnp.float32),
        pltpu.VMEM((ROWS_PER_TILE,), jnp.float32),
    ],
    compiler_params=pltpu.CompilerParams(needs_layout_passes=False),
)
def _deg_kernel(dst_hbm, out0, out1, sh, dst_v, hist, red_a, red_b):
    c = lax.axis_index("c")
    s = lax.axis_index("s")
    wid = c * NSUB + s
    pltpu.sync_copy(dst_hbm.at[pl.ds(wid * NBT, NBT)], dst_v)

    zero16 = jnp.zeros((16,), jnp.float32)
    one16 = jnp.ones((16,), jnp.float32)

    def zstep(i, carry):
        hist[pl.ds(i * 16, 16)] = zero16
        return carry

    lax.fori_loop(0, NP // 16, zstep, 0)

    def hstep(j, carry):
        for k in range(SBATCH // 16):
            idx = dst_v[j, pl.ds(k * 16, 16)]
            plsc.addupdate_scatter(hist, [idx], one16)
        return carry

    lax.fori_loop(0, NBT, hstep, 0)

    pltpu.sync_copy(hist, sh.at[s])
    plsc.subcore_barrier()

    sl = pl.ds(ROWS_PER_TILE * s, ROWS_PER_TILE)
    pltpu.sync_copy(sh.at[0].at[sl], red_a)
    for k in range(1, NSUB):
        pltpu.sync_copy(sh.at[k].at[sl], red_b)

        def astep(m, carry):
            red_a[pl.ds(m * 16, 16)] = (red_a[pl.ds(m * 16, 16)]
                                        + red_b[pl.ds(m * 16, 16)])
            return carry

        lax.fori_loop(0, ROWS_PER_TILE // 16, astep, 0)

    @pl.when(c == 0)
    def _():
        pltpu.sync_copy(red_a, out0.at[sl])

    @pl.when(c == 1)
    def _():
        pltpu.sync_copy(red_a, out1.at[sl])


def _make_agg(feat):
    @functools.partial(
        pl.kernel,
        out_type=jax.ShapeDtypeStruct((NCORES, NP, feat), jnp.float32),
        mesh=_MESH,
        scratch_types=[
            pltpu.VMEM_SHARED((NP, feat), jnp.float32),
            pltpu.VMEM((ST, SBATCH), jnp.int32),
            pltpu.VMEM((ST, SBATCH), jnp.int32),
            [pltpu.VMEM((SBATCH, feat), jnp.float32) for _ in range(NBUF)],
            [pltpu.SemaphoreType.DMA for _ in range(NBUF)],
        ],
    )
    def agg(x_hbm, src_hbm, dst_hbm, out_hbm,
            acc, src_v, dst_v, bufs, sems):
        c = lax.axis_index("c")
        s = lax.axis_index("s")

        zero16 = jnp.zeros((16,), jnp.float32)

        def zfill(i, carry):
            r = i // (feat // 16)
            k = i % (feat // 16)
            bufs[0][r, pl.ds(k * 16, 16)] = zero16
            return carry

        lax.fori_loop(0, SBATCH * feat // 16, zfill, 0)
        for r in range(ROWS_PER_TILE // SBATCH):
            pltpu.sync_copy(
                bufs[0],
                acc.at[pl.ds(ROWS_PER_TILE * s + r * SBATCH, SBATCH)])
        plsc.subcore_barrier()

        nst = jnp.where(c == 0, NST0, NST1)
        row0 = jnp.where(c == 0, ST * NST0 * s,
                         ST * NST0 * NSUB + ST * NST1 * s)

        def stage(h, carry):
            base = row0 + h * ST
            pltpu.sync_copy(src_hbm.at[pl.ds(base, ST)], src_v)
            pltpu.sync_copy(dst_hbm.at[pl.ds(base, ST)], dst_v)
            for q in range(NBUF - 1):
                pltpu.async_copy(x_hbm.at[src_v.at[q]], bufs[q], sems[q])

            def step(i, carry2):
                for q in range(NBUF):
                    j = NBUF * i + q
                    pltpu.make_async_copy(
                        x_hbm.at[src_v.at[j]], bufs[q], sems[q]).wait()
                    pltpu.sync_copy(bufs[q], acc.at[dst_v.at[j]], add=True)
                    qn = (q + NBUF - 1) % NBUF

                    @pl.when(j + NBUF - 1 < ST)
                    def _():
                        pltpu.async_copy(
                            x_hbm.at[src_v.at[j + NBUF - 1]],
                            bufs[qn], sems[qn])
                return carry2

            lax.fori_loop(0, ST // NBUF, step, 0)
            return carry

        lax.fori_loop(0, nst, stage, 0)
        plsc.subcore_barrier()
        sl = pl.ds(ROWS_PER_TILE * s, ROWS_PER_TILE)
        pltpu.sync_copy(acc.at[sl], out_hbm.at[c].at[sl])

    return agg


_agg128 = _make_agg(128)


BLK = 1024
GRID = NP // BLK


def _scale_body(deg0_ref, deg1_ref, x_ref, dinv_ref, xt_ref):
    d = deg0_ref[...] + deg1_ref[...] + 1.0
    di = lax.rsqrt(d)
    dinv_ref[...] = di
    xt_ref[...] = x_ref[...] * di


def _scale_call(deg0, deg1, x_pad):
    return pl.pallas_call(
        _scale_body,
        grid=(GRID,),
        in_specs=[
            pl.BlockSpec((BLK, 1), lambda i: (i, 0)),
            pl.BlockSpec((BLK, 1), lambda i: (i, 0)),
            pl.BlockSpec((BLK, 128), lambda i: (i, 0)),
        ],
        out_specs=[
            pl.BlockSpec((BLK, 1), lambda i: (i, 0)),
            pl.BlockSpec((BLK, 128), lambda i: (i, 0)),
        ],
        out_shape=[
            jax.ShapeDtypeStruct((NP, 1), jnp.float32),
            jax.ShapeDtypeStruct((NP, 128), jnp.float32),
        ],
    )(deg0, deg1, x_pad)


def _fused_body(acc_ref, xt_ref, dinv_ref, w1_ref, b1_ref, w2_ref, out_ref):
    di = dinv_ref[...]
    z = (acc_ref[0] + acc_ref[1] + xt_ref[...]) * di
    h = jnp.dot(z, w1_ref[...], preferred_element_type=jnp.float32)
    h = jnp.maximum(h + b1_ref[...], 0.0)
    t = jnp.dot(h, w2_ref[...], preferred_element_type=jnp.float32)
    out_ref[...] = jnp.concatenate(
        [t * di, jnp.zeros((t.shape[0], 64), jnp.float32)], axis=1)


def _fused_call(acc1, xt, dinv, W1, b1, W2):
    return pl.pallas_call(
        _fused_body,
        grid=(GRID,),
        in_specs=[
            pl.BlockSpec((NCORES, BLK, 128), lambda i: (0, i, 0)),
            pl.BlockSpec((BLK, 128), lambda i: (i, 0)),
            pl.BlockSpec((BLK, 1), lambda i: (i, 0)),
            pl.BlockSpec((128, 1024), lambda i: (0, 0)),
            pl.BlockSpec((1, 1024), lambda i: (0, 0)),
            pl.BlockSpec((1024, 64), lambda i: (0, 0)),
        ],
        out_specs=pl.BlockSpec((BLK, 128), lambda i: (i, 0)),
        out_shape=jax.ShapeDtypeStruct((NP, 128), jnp.float32),
    )(acc1, xt, dinv, W1, b1.reshape(1, 1024), W2)


def _softmax_body(acc_ref, tt_ref, dinv_ref, b2_ref, out_ref):
    z128 = (acc_ref[0] + acc_ref[1] + tt_ref[...]) * dinv_ref[...]
    z = z128[:, :64] + b2_ref[...]
    m = jnp.max(z, axis=1, keepdims=True)
    e = jnp.exp(z - m)
    out_ref[...] = e / jnp.sum(e, axis=1, keepdims=True)


def _softmax_call(acc2, tt, dinv, b2):
    return pl.pallas_call(
        _softmax_body,
        grid=(GRID,),
        in_specs=[
            pl.BlockSpec((NCORES, BLK, 128), lambda i: (0, i, 0)),
            pl.BlockSpec((BLK, 128), lambda i: (i, 0)),
            pl.BlockSpec((BLK, 1), lambda i: (i, 0)),
            pl.BlockSpec((1, 64), lambda i: (0, 0)),
        ],
        out_specs=pl.BlockSpec((BLK, 64), lambda i: (i, 0)),
        out_shape=jax.ShapeDtypeStruct((NP, 64), jnp.float32),
    )(acc2, tt, dinv, b2.reshape(1, 64))


def kernel(x, edge_index, W1, b1, W2, b2):
    n = x.shape[0]
    e = edge_index.shape[1]
    src = edge_index[0].astype(jnp.int32)
    dst = edge_index[1].astype(jnp.int32)
    pad = EPAD - e
    pad_idx = n + (jnp.arange(pad, dtype=jnp.int32) % (NP - n))
    src_p = jnp.concatenate([src, pad_idx])
    dst_p = jnp.concatenate([dst, pad_idx])
    src64 = src_p.reshape(EPAD // SBATCH, SBATCH)
    dst64 = dst_p.reshape(EPAD // SBATCH, SBATCH)
    x_pad = jnp.zeros((NP, 128), jnp.float32).at[:n].set(x)

    deg0, deg1 = _deg_kernel(dst64)
    dinv, xt = _scale_call(deg0.reshape(NP, 1), deg1.reshape(NP, 1), x_pad)
    acc1 = _agg128(xt, src64, dst64)
    tt = _fused_call(acc1, xt, dinv, W1, b1, W2)
    acc2 = _agg128(tt, src64, dst64)
    out = _softmax_call(acc2, tt, dinv, b2)
    return out[:n]
